# baseline (speedup 1.0000x reference)
import functools
import math

import jax
import jax.numpy as jnp
from jax import lax
from jax.experimental import pallas as pl
from jax.experimental.pallas import tpu as pltpu
from jax.experimental.pallas import tpu_sc as plsc

ALPHA = 0.1
THETA = math.log(0.5 / 1.0 + 1.0)

NC = 2
NS = 16
NW = NC * NS
CHUNK = 128


def _pre_kernel(x_ref, u_ref, Wf_ref, bf_ref, ecol_ref, div_ref, We0_ref,
                Wsin_ref, Wcos_ref, beig_ref, Wdec_ref, bdec_ref, swr_ref,
                h_ref, S_ref, acc_ref):
    i = pl.program_id(0)
    nsteps = pl.num_programs(0)
    h = jnp.dot(x_ref[...], Wf_ref[...], preferred_element_type=jnp.float32)
    h = h + bf_ref[...]
    h_ref[...] = h

    @pl.when(i == 0)
    def _():
        acc_ref[...] = jnp.zeros_like(acc_ref)

    acc_ref[...] += lax.dot_general(
        u_ref[...], h, (((0,), (0,)), ((), ())),
        preferred_element_type=jnp.float32)

    @pl.when(i == nsteps - 1)
    def _():
        ecol = ecol_ref[...]
        pe = (ecol * 100.0) * div_ref[...]
        eig = (ecol * We0_ref[...]
               + jnp.dot(jnp.sin(pe), Wsin_ref[...],
                         preferred_element_type=jnp.float32)
               + jnp.dot(jnp.cos(pe), Wcos_ref[...],
                         preferred_element_type=jnp.float32)
               + beig_ref[...])
        new_e = jnp.dot(eig, Wdec_ref[...],
                        preferred_element_type=jnp.float32) + bdec_ref[...]
        M = new_e.shape[1]
        combo = new_e[:, 0:1] * swr_ref[0:1, :]
        for m in range(1, M):
            combo = combo + new_e[:, m:m + 1] * swr_ref[m:m + 1, :]
        S_ref[...] = acc_ref[...] * combo


def _hspec_kernel(h_ref, u_ref, S_ref, sw0_ref, out_ref):
    out_ref[...] = (sw0_ref[...] * h_ref[...]
                    + jnp.dot(u_ref[...], S_ref[...],
                              preferred_element_type=jnp.float32))


def _post_kernel(h_ref, hip_ref, degp_ref, Wgc_ref, W1_ref, b1_ref, W2_ref,
                 b2_ref, out_ref):
    hi = hip_ref[0] + hip_ref[1]
    deg = degp_ref[0, :, 0:1] + degp_ref[1, :, 0:1]
    hi = hi / jnp.maximum(deg, 1.0)
    support = (1.0 - ALPHA) * hi + ALPHA * h_ref[...]
    out = THETA * jnp.dot(support, Wgc_ref[...],
                          preferred_element_type=jnp.float32) \
        + (1.0 - THETA) * support
    out = jnp.maximum(
        jnp.dot(out, W1_ref[...], preferred_element_type=jnp.float32)
        + b1_ref[...], 0.0)
    out_ref[...] = jnp.dot(out, W2_ref[...],
                           preferred_element_type=jnp.float32) + b2_ref[...]


def _make_sc_spmm(n_acc, d, chunks, rows_pt):
    nz = rows_pt // CHUNK
    GD = 4
    assert chunks % 2 == 0 and chunks % GD == 0
    mesh = plsc.VectorSubcoreMesh(core_axis_name="c", subcore_axis_name="s")

    @functools.partial(
        pl.kernel,
        out_type=[
            jax.ShapeDtypeStruct((NC * n_acc, d), jnp.float32),
            jax.ShapeDtypeStruct((NC * n_acc, d), jnp.float32),
        ],
        mesh=mesh,
        scratch_types=[
            pltpu.VMEM((CHUNK,), jnp.int32),
            pltpu.VMEM((CHUNK,), jnp.int32),
            pltpu.VMEM((chunks, 1, CHUNK), jnp.int32),
            pltpu.VMEM((nz, 1, CHUNK), jnp.int32),
            pltpu.VMEM((CHUNK, d), jnp.float32),
            pltpu.VMEM((CHUNK, d), jnp.float32),
            pltpu.VMEM_SHARED((n_acc, d), jnp.float32),
            pltpu.SemaphoreType.DMA,
            pltpu.SemaphoreType.DMA,
            pltpu.SemaphoreType.DMA,
            pltpu.SemaphoreType.DMA,
        ],
    )
    def spmm(src3_hbm, dst3_hbm, hspec_hbm, zeros_hbm, ones_hbm, rowidx_hbm,
             hi_out, deg_out, srcvA, srcvB, dstv3, idx5, rowsA, rowsB, acc,
             sem, semgA, semgB, semi):
        cid = lax.axis_index("c")
        sid = lax.axis_index("s")
        wid = sid * NC + cid
        row0 = sid * rows_pt
        out0 = cid * n_acc + row0


        pltpu.sync_copy(dst3_hbm.at[wid], dstv3)
        pltpu.sync_copy(rowidx_hbm.at[sid], idx5)

        def zero_acc():
            pltpu.sync_copy(zeros_hbm, rowsA)
            for z in range(nz):
                pltpu.sync_copy(rowsA, acc.at[idx5.at[z, 0]])
            plsc.subcore_barrier()

        def copy_out(dst_hbm_out):
            for z in range(nz):
                pltpu.sync_copy(acc.at[idx5.at[z, 0]], rowsA)
                pltpu.sync_copy(rowsA, dst_hbm_out.at[pl.ds(out0 + z * CHUNK,
                                                            CHUNK)])
            plsc.subcore_barrier()

        zero_acc()
        pltpu.sync_copy(src3_hbm.at[wid, 0, 0], srcvA)
        pltpu.sync_copy(src3_hbm.at[wid, 1, 0], srcvB)
        pltpu.async_copy(hspec_hbm.at[srcvA], rowsA, semgA)

        def step(i, carry):
            j = 2 * i
            pltpu.async_copy(hspec_hbm.at[srcvB], rowsB, semgB)
            pltpu.make_async_copy(hspec_hbm.at[srcvA], rowsA, semgA).wait()
            di = pltpu.async_copy(src3_hbm.at[wid, j + 2, 0], srcvA, semi)
            pltpu.sync_copy(rowsA, acc.at[dstv3.at[j, 0]], add=True)
            di.wait()
            pltpu.async_copy(hspec_hbm.at[srcvA], rowsA, semgA)
            pltpu.make_async_copy(hspec_hbm.at[srcvB], rowsB, semgB).wait()
            di = pltpu.async_copy(src3_hbm.at[wid, j + 3, 0], srcvB, semi)
            pltpu.sync_copy(rowsB, acc.at[dstv3.at[j + 1, 0]], add=True)
            di.wait()
            return carry

        lax.fori_loop(0, chunks // 2, step, 0)
        pltpu.make_async_copy(hspec_hbm.at[srcvA], rowsA, semgA).wait()
        plsc.subcore_barrier()
        copy_out(hi_out)

        zero_acc()
        pltpu.sync_copy(ones_hbm, rowsA)

        def dstep(j, carry):
            pltpu.sync_copy(rowsA, acc.at[dstv3.at[j, 0]], add=True)
            return carry

        lax.fori_loop(0, chunks, dstep, 0)
        plsc.subcore_barrier()
        copy_out(deg_out)

    return spmm


def kernel(x, e, u, edge_index, W_feat, b_feat, W_eig, b_eig, W_dec, b_dec,
           spec_weight, W_gc, W1, b1, W2, b2):
    n, d = x.shape
    k = e.shape[0]
    hid = W_eig.shape[1]
    m = W_dec.shape[1]
    num_e = edge_index.shape[1]
    nclass = W2.shape[1]
    hdim = W1.shape[1]

    blk = 1000
    nblk = n // blk

    ecol = e.reshape(k, 1)
    div = jnp.exp(jnp.arange(0, hid, 2, dtype=jnp.float32)
                  * (-math.log(10000.0) / hid)).reshape(1, hid // 2)
    We0 = W_eig[0].reshape(1, hid)
    Wsin = W_eig[1:1 + hid // 2]
    Wcos = W_eig[1 + hid // 2:]
    sw0 = spec_weight[0, 0].reshape(1, d)
    swr = spec_weight[0, 1:]

    f32 = jnp.float32
    h, S = pl.pallas_call(
        _pre_kernel,
        grid=(nblk,),
        in_specs=[
            pl.BlockSpec((blk, d), lambda i: (i, 0)),
            pl.BlockSpec((blk, k), lambda i: (i, 0)),
            pl.BlockSpec((d, d), lambda i: (0, 0)),
            pl.BlockSpec((1, d), lambda i: (0, 0)),
            pl.BlockSpec((k, 1), lambda i: (0, 0)),
            pl.BlockSpec((1, hid // 2), lambda i: (0, 0)),
            pl.BlockSpec((1, hid), lambda i: (0, 0)),
            pl.BlockSpec((hid // 2, hid), lambda i: (0, 0)),
            pl.BlockSpec((hid // 2, hid), lambda i: (0, 0)),
            pl.BlockSpec((1, hid), lambda i: (0, 0)),
            pl.BlockSpec((hid, m), lambda i: (0, 0)),
            pl.BlockSpec((1, m), lambda i: (0, 0)),
            pl.BlockSpec((m, d), lambda i: (0, 0)),
        ],
        out_specs=[
            pl.BlockSpec((blk, d), lambda i: (i, 0)),
            pl.BlockSpec((k, d), lambda i: (0, 0)),
        ],
        out_shape=[
            jax.ShapeDtypeStruct((n, d), f32),
            jax.ShapeDtypeStruct((k, d), f32),
        ],
        scratch_shapes=[pltpu.VMEM((k, d), f32)],
    )(x, u, W_feat, b_feat.reshape(1, d), ecol, div, We0, Wsin, Wcos,
      b_eig.reshape(1, hid), W_dec, b_dec.reshape(1, m), swr)

    h_spec = pl.pallas_call(
        _hspec_kernel,
        grid=(nblk,),
        in_specs=[
            pl.BlockSpec((blk, d), lambda i: (i, 0)),
            pl.BlockSpec((blk, k), lambda i: (i, 0)),
            pl.BlockSpec((k, d), lambda i: (0, 0)),
            pl.BlockSpec((1, d), lambda i: (0, 0)),
        ],
        out_specs=pl.BlockSpec((blk, d), lambda i: (i, 0)),
        out_shape=jax.ShapeDtypeStruct((n, d), f32),
    )(h, u, S, sw0)

    chunks = -(-num_e // (NW * CHUNK))
    chunks = (chunks + 3) // 4 * 4
    e_pad = NW * chunks * CHUNK
    rows_pt = ((-(-(n + 1) // NS)) + CHUNK - 1) // CHUNK * CHUNK
    n_acc = rows_pt * NS
    src = jnp.concatenate(
        [edge_index[0], jnp.zeros((e_pad - num_e,), jnp.int32)])
    dst = jnp.concatenate(
        [edge_index[1], jnp.full((e_pad - num_e,), n, jnp.int32)])
    src3 = jnp.concatenate(
        [src.reshape(NW, chunks, CHUNK),
         jnp.zeros((NW, 2, CHUNK), jnp.int32)],
        axis=1).reshape(NW, chunks + 2, 1, CHUNK)
    dst3 = dst.reshape(NW, chunks, 1, CHUNK)
    zeros_hbm = jnp.zeros((CHUNK, d), f32)
    ones_hbm = jnp.ones((CHUNK, d), f32)
    rowidx_hbm = jnp.arange(n_acc, dtype=jnp.int32).reshape(
        NS, rows_pt // CHUNK, 1, CHUNK)

    spmm = _make_sc_spmm(n_acc, d, chunks, rows_pt)
    hi_p, deg_p = spmm(src3, dst3, h_spec, zeros_hbm, ones_hbm, rowidx_hbm)
    hi_p = hi_p.reshape(NC, n_acc, d)
    deg_p = deg_p.reshape(NC, n_acc, d)

    logits = pl.pallas_call(
        _post_kernel,
        grid=(nblk,),
        in_specs=[
            pl.BlockSpec((blk, d), lambda i: (i, 0)),
            pl.BlockSpec((NC, blk, d), lambda i: (0, i, 0)),
            pl.BlockSpec((NC, blk, d), lambda i: (0, i, 0)),
            pl.BlockSpec((d, d), lambda i: (0, 0)),
            pl.BlockSpec((d, hdim), lambda i: (0, 0)),
            pl.BlockSpec((1, hdim), lambda i: (0, 0)),
            pl.BlockSpec((hdim, nclass), lambda i: (0, 0)),
            pl.BlockSpec((1, nclass), lambda i: (0, 0)),
        ],
        out_specs=pl.BlockSpec((blk, nclass), lambda i: (i, 0)),
        out_shape=jax.ShapeDtypeStruct((n, nclass), f32),
    )(h, hi_p, deg_p, W_gc, W1, b1.reshape(1, hdim), W2, b2.reshape(1, nclass))

    return logits


# speedup vs baseline: 1.0202x; 1.0202x over previous
import functools
import math

import jax
import jax.numpy as jnp
from jax import lax
from jax.experimental import pallas as pl
from jax.experimental.pallas import tpu as pltpu
from jax.experimental.pallas import tpu_sc as plsc

ALPHA = 0.1
THETA = math.log(0.5 / 1.0 + 1.0)

NC = 2
NS = 16
NW = NC * NS
CHUNK = 128


def _pre_kernel(x_ref, u_ref, Wf_ref, bf_ref, ecol_ref, div_ref, We0_ref,
                Wsin_ref, Wcos_ref, beig_ref, Wdec_ref, bdec_ref, swr_ref,
                h_ref, S_ref, acc_ref):
    i = pl.program_id(0)
    nsteps = pl.num_programs(0)
    h = jnp.dot(x_ref[...], Wf_ref[...], preferred_element_type=jnp.float32)
    h = h + bf_ref[...]
    h_ref[...] = h

    @pl.when(i == 0)
    def _():
        acc_ref[...] = jnp.zeros_like(acc_ref)

    acc_ref[...] += lax.dot_general(
        u_ref[...], h, (((0,), (0,)), ((), ())),
        preferred_element_type=jnp.float32)

    @pl.when(i == nsteps - 1)
    def _():
        ecol = ecol_ref[...]
        pe = (ecol * 100.0) * div_ref[...]
        eig = (ecol * We0_ref[...]
               + jnp.dot(jnp.sin(pe), Wsin_ref[...],
                         preferred_element_type=jnp.float32)
               + jnp.dot(jnp.cos(pe), Wcos_ref[...],
                         preferred_element_type=jnp.float32)
               + beig_ref[...])
        new_e = jnp.dot(eig, Wdec_ref[...],
                        preferred_element_type=jnp.float32) + bdec_ref[...]
        M = new_e.shape[1]
        combo = new_e[:, 0:1] * swr_ref[0:1, :]
        for m in range(1, M):
            combo = combo + new_e[:, m:m + 1] * swr_ref[m:m + 1, :]
        S_ref[...] = acc_ref[...] * combo


def _hspec_kernel(h_ref, u_ref, S_ref, sw0_ref, out_ref):
    out_ref[...] = (sw0_ref[...] * h_ref[...]
                    + jnp.dot(u_ref[...], S_ref[...],
                              preferred_element_type=jnp.float32))


def _post_kernel(h_ref, hip_ref, degp_ref, Wgc_ref, W1_ref, b1_ref, W2_ref,
                 b2_ref, out_ref):
    hi = hip_ref[0] + hip_ref[1]
    deg = degp_ref[0, :, 0:1] + degp_ref[1, :, 0:1]
    hi = hi / jnp.maximum(deg, 1.0)
    support = (1.0 - ALPHA) * hi + ALPHA * h_ref[...]
    out = THETA * jnp.dot(support, Wgc_ref[...],
                          preferred_element_type=jnp.float32) \
        + (1.0 - THETA) * support
    out = jnp.maximum(
        jnp.dot(out, W1_ref[...], preferred_element_type=jnp.float32)
        + b1_ref[...], 0.0)
    out_ref[...] = jnp.dot(out, W2_ref[...],
                           preferred_element_type=jnp.float32) + b2_ref[...]


def _make_sc_spmm(n_acc, d, chunks, rows_pt):
    epw = chunks * CHUNK
    mesh = plsc.VectorSubcoreMesh(core_axis_name="c", subcore_axis_name="s")

    @functools.partial(
        pl.kernel,
        out_type=[
            jax.ShapeDtypeStruct((NC * n_acc, d), jnp.float32),
            jax.ShapeDtypeStruct((NC * n_acc, d), jnp.float32),
        ],
        mesh=mesh,
        scratch_types=[
            pltpu.VMEM((CHUNK,), jnp.int32),
            pltpu.VMEM((CHUNK,), jnp.int32),
            pltpu.VMEM((CHUNK,), jnp.int32),
            pltpu.VMEM((CHUNK,), jnp.int32),
            pltpu.VMEM((CHUNK, d), jnp.float32),
            pltpu.VMEM((CHUNK, d), jnp.float32),
            pltpu.VMEM_SHARED((n_acc, d), jnp.float32),
            pltpu.SemaphoreType.DMA,
            pltpu.SemaphoreType.DMA,
            pltpu.SemaphoreType.DMA,
        ],
    )
    def spmm(src_hbm, dst_hbm, hspec_hbm, zeros_hbm, ones_hbm, rowidx_hbm,
             hi_out, deg_out, srcv, dstv, srcvB, dstvB, rows, rowsB, acc,
             semA, semB, semi):
        cid = lax.axis_index("c")
        sid = lax.axis_index("s")
        wid = sid * NC + cid
        row0 = sid * rows_pt
        base = wid * epw
        out0 = cid * n_acc + row0
        nz = rows_pt // CHUNK


        def zero_acc():
            pltpu.sync_copy(zeros_hbm, rows)
            for z in range(nz):
                pltpu.sync_copy(rowidx_hbm.at[pl.ds(row0 + z * CHUNK, CHUNK)],
                                dstv)
                pltpu.sync_copy(rows, acc.at[dstv])
            plsc.subcore_barrier()

        def copy_out(dst_hbm_out):
            for z in range(nz):
                pltpu.sync_copy(rowidx_hbm.at[pl.ds(row0 + z * CHUNK, CHUNK)],
                                dstv)
                pltpu.sync_copy(acc.at[dstv], rows)
                pltpu.sync_copy(rows, dst_hbm_out.at[pl.ds(out0 + z * CHUNK,
                                                           CHUNK)])
            plsc.subcore_barrier()

        zero_acc()
        pltpu.sync_copy(src_hbm.at[pl.ds(base, CHUNK)], srcv)
        pltpu.sync_copy(dst_hbm.at[pl.ds(base, CHUNK)], dstv)
        pltpu.async_copy(hspec_hbm.at[srcv], rows, semA)
        pltpu.async_copy(src_hbm.at[pl.ds(base + CHUNK, CHUNK)], srcvB, semi)
        pltpu.async_copy(dst_hbm.at[pl.ds(base + CHUNK, CHUNK)], dstvB, semi)

        def step(i, carry):
            j = 2 * i
            pltpu.make_async_copy(src_hbm.at[pl.ds(base, CHUNK)], srcvB,
                                  semi).wait()
            pltpu.make_async_copy(dst_hbm.at[pl.ds(base, CHUNK)], dstvB,
                                  semi).wait()
            pltpu.async_copy(hspec_hbm.at[srcvB], rowsB, semB)
            pltpu.make_async_copy(hspec_hbm.at[srcv], rows, semA).wait()
            pltpu.sync_copy(rows, acc.at[dstv], add=True)
            off2 = base + (j + 2) * CHUNK
            pltpu.async_copy(src_hbm.at[pl.ds(off2, CHUNK)], srcv, semi)
            pltpu.async_copy(dst_hbm.at[pl.ds(off2, CHUNK)], dstv, semi)
            pltpu.make_async_copy(src_hbm.at[pl.ds(base, CHUNK)], srcv,
                                  semi).wait()
            pltpu.make_async_copy(dst_hbm.at[pl.ds(base, CHUNK)], dstv,
                                  semi).wait()
            pltpu.async_copy(hspec_hbm.at[srcv], rows, semA)
            pltpu.make_async_copy(hspec_hbm.at[srcvB], rowsB, semB).wait()
            pltpu.sync_copy(rowsB, acc.at[dstvB], add=True)
            off3 = base + (j + 3) * CHUNK
            pltpu.async_copy(src_hbm.at[pl.ds(off3, CHUNK)], srcvB, semi)
            pltpu.async_copy(dst_hbm.at[pl.ds(off3, CHUNK)], dstvB, semi)
            return carry

        lax.fori_loop(0, chunks // 2, step, 0)
        pltpu.make_async_copy(hspec_hbm.at[srcv], rows, semA).wait()
        pltpu.make_async_copy(src_hbm.at[pl.ds(base, CHUNK)], srcvB,
                              semi).wait()
        pltpu.make_async_copy(dst_hbm.at[pl.ds(base, CHUNK)], dstvB,
                              semi).wait()
        plsc.subcore_barrier()
        copy_out(hi_out)

        zero_acc()
        pltpu.sync_copy(ones_hbm, rows)

        def dstep(j, carry):
            off = base + j * CHUNK
            pltpu.sync_copy(dst_hbm.at[pl.ds(off, CHUNK)], dstv)
            pltpu.sync_copy(rows, acc.at[dstv], add=True)
            return carry

        lax.fori_loop(0, chunks, dstep, 0)
        plsc.subcore_barrier()
        copy_out(deg_out)

    return spmm


def kernel(x, e, u, edge_index, W_feat, b_feat, W_eig, b_eig, W_dec, b_dec,
           spec_weight, W_gc, W1, b1, W2, b2):
    n, d = x.shape
    k = e.shape[0]
    hid = W_eig.shape[1]
    m = W_dec.shape[1]
    num_e = edge_index.shape[1]
    nclass = W2.shape[1]
    hdim = W1.shape[1]

    blk = 1000
    nblk = n // blk

    ecol = e.reshape(k, 1)
    div = jnp.exp(jnp.arange(0, hid, 2, dtype=jnp.float32)
                  * (-math.log(10000.0) / hid)).reshape(1, hid // 2)
    We0 = W_eig[0].reshape(1, hid)
    Wsin = W_eig[1:1 + hid // 2]
    Wcos = W_eig[1 + hid // 2:]
    sw0 = spec_weight[0, 0].reshape(1, d)
    swr = spec_weight[0, 1:]

    f32 = jnp.float32
    h, S = pl.pallas_call(
        _pre_kernel,
        grid=(nblk,),
        in_specs=[
            pl.BlockSpec((blk, d), lambda i: (i, 0)),
            pl.BlockSpec((blk, k), lambda i: (i, 0)),
            pl.BlockSpec((d, d), lambda i: (0, 0)),
            pl.BlockSpec((1, d), lambda i: (0, 0)),
            pl.BlockSpec((k, 1), lambda i: (0, 0)),
            pl.BlockSpec((1, hid // 2), lambda i: (0, 0)),
            pl.BlockSpec((1, hid), lambda i: (0, 0)),
            pl.BlockSpec((hid // 2, hid), lambda i: (0, 0)),
            pl.BlockSpec((hid // 2, hid), lambda i: (0, 0)),
            pl.BlockSpec((1, hid), lambda i: (0, 0)),
            pl.BlockSpec((hid, m), lambda i: (0, 0)),
            pl.BlockSpec((1, m), lambda i: (0, 0)),
            pl.BlockSpec((m, d), lambda i: (0, 0)),
        ],
        out_specs=[
            pl.BlockSpec((blk, d), lambda i: (i, 0)),
            pl.BlockSpec((k, d), lambda i: (0, 0)),
        ],
        out_shape=[
            jax.ShapeDtypeStruct((n, d), f32),
            jax.ShapeDtypeStruct((k, d), f32),
        ],
        scratch_shapes=[pltpu.VMEM((k, d), f32)],
    )(x, u, W_feat, b_feat.reshape(1, d), ecol, div, We0, Wsin, Wcos,
      b_eig.reshape(1, hid), W_dec, b_dec.reshape(1, m), swr)

    h_spec = pl.pallas_call(
        _hspec_kernel,
        grid=(nblk,),
        in_specs=[
            pl.BlockSpec((blk, d), lambda i: (i, 0)),
            pl.BlockSpec((blk, k), lambda i: (i, 0)),
            pl.BlockSpec((k, d), lambda i: (0, 0)),
            pl.BlockSpec((1, d), lambda i: (0, 0)),
        ],
        out_specs=pl.BlockSpec((blk, d), lambda i: (i, 0)),
        out_shape=jax.ShapeDtypeStruct((n, d), f32),
    )(h, u, S, sw0)

    chunks = -(-num_e // (NW * CHUNK))
    chunks = (chunks + 1) // 2 * 2
    e_pad = NW * chunks * CHUNK
    rows_pt = ((-(-(n + 1) // NS)) + CHUNK - 1) // CHUNK * CHUNK
    n_acc = rows_pt * NS
    src = jnp.concatenate(
        [edge_index[0], jnp.zeros((e_pad + 2 * CHUNK - num_e,), jnp.int32)])
    dst = jnp.concatenate(
        [edge_index[1], jnp.full((e_pad - num_e,), n, jnp.int32),
         jnp.zeros((2 * CHUNK,), jnp.int32)])
    zeros_hbm = jnp.zeros((CHUNK, d), f32)
    ones_hbm = jnp.ones((CHUNK, d), f32)
    rowidx_hbm = jnp.arange(n_acc, dtype=jnp.int32)

    spmm = _make_sc_spmm(n_acc, d, chunks, rows_pt)
    hi_p, deg_p = spmm(src, dst, h_spec, zeros_hbm, ones_hbm, rowidx_hbm)
    hi_p = hi_p.reshape(NC, n_acc, d)
    deg_p = deg_p.reshape(NC, n_acc, d)

    logits = pl.pallas_call(
        _post_kernel,
        grid=(nblk,),
        in_specs=[
            pl.BlockSpec((blk, d), lambda i: (i, 0)),
            pl.BlockSpec((NC, blk, d), lambda i: (0, i, 0)),
            pl.BlockSpec((NC, blk, d), lambda i: (0, i, 0)),
            pl.BlockSpec((d, d), lambda i: (0, 0)),
            pl.BlockSpec((d, hdim), lambda i: (0, 0)),
            pl.BlockSpec((1, hdim), lambda i: (0, 0)),
            pl.BlockSpec((hdim, nclass), lambda i: (0, 0)),
            pl.BlockSpec((1, nclass), lambda i: (0, 0)),
        ],
        out_specs=pl.BlockSpec((blk, nclass), lambda i: (i, 0)),
        out_shape=jax.ShapeDtypeStruct((n, nclass), f32),
    )(h, hi_p, deg_p, W_gc, W1, b1.reshape(1, hdim), W2, b2.reshape(1, nclass))

    return logits


# speedup vs baseline: 1.4119x; 1.3839x over previous
import functools
import math

import jax
import jax.numpy as jnp
from jax import lax
from jax.experimental import pallas as pl
from jax.experimental.pallas import tpu as pltpu
from jax.experimental.pallas import tpu_sc as plsc

ALPHA = 0.1
THETA = math.log(0.5 / 1.0 + 1.0)

NC = 2
NS = 16
NW = NC * NS
CHUNK = 128


def _pre_kernel(x_ref, u_ref, Wf_ref, bf_ref, ecol_ref, div_ref, We0_ref,
                Wsin_ref, Wcos_ref, beig_ref, Wdec_ref, bdec_ref, swr_ref,
                h_ref, S_ref, acc_ref):
    i = pl.program_id(0)
    nsteps = pl.num_programs(0)
    h = jnp.dot(x_ref[...], Wf_ref[...], preferred_element_type=jnp.float32)
    h = h + bf_ref[...]
    h_ref[...] = h

    @pl.when(i == 0)
    def _():
        acc_ref[...] = jnp.zeros_like(acc_ref)

    acc_ref[...] += lax.dot_general(
        u_ref[...], h, (((0,), (0,)), ((), ())),
        preferred_element_type=jnp.float32)

    @pl.when(i == nsteps - 1)
    def _():
        ecol = ecol_ref[...]
        pe = (ecol * 100.0) * div_ref[...]
        eig = (ecol * We0_ref[...]
               + jnp.dot(jnp.sin(pe), Wsin_ref[...],
                         preferred_element_type=jnp.float32)
               + jnp.dot(jnp.cos(pe), Wcos_ref[...],
                         preferred_element_type=jnp.float32)
               + beig_ref[...])
        new_e = jnp.dot(eig, Wdec_ref[...],
                        preferred_element_type=jnp.float32) + bdec_ref[...]
        M = new_e.shape[1]
        combo = new_e[:, 0:1] * swr_ref[0:1, :]
        for m in range(1, M):
            combo = combo + new_e[:, m:m + 1] * swr_ref[m:m + 1, :]
        S_ref[...] = acc_ref[...] * combo


def _hspec_kernel(h_ref, u_ref, S_ref, sw0_ref, out_ref):
    out_ref[...] = (sw0_ref[...] * h_ref[...]
                    + jnp.dot(u_ref[...], S_ref[...],
                              preferred_element_type=jnp.float32))


def _post_kernel(h_ref, hip_ref, degp_ref, Wgc_ref, W1_ref, b1_ref, W2_ref,
                 b2_ref, out_ref):
    hi = hip_ref[0] + hip_ref[1]
    deg = degp_ref[0, :, 0:1] + degp_ref[1, :, 0:1]
    hi = hi / jnp.maximum(deg, 1.0)
    support = (1.0 - ALPHA) * hi + ALPHA * h_ref[...]
    out = THETA * jnp.dot(support, Wgc_ref[...],
                          preferred_element_type=jnp.float32) \
        + (1.0 - THETA) * support
    out = jnp.maximum(
        jnp.dot(out, W1_ref[...], preferred_element_type=jnp.float32)
        + b1_ref[...], 0.0)
    out_ref[...] = jnp.dot(out, W2_ref[...],
                           preferred_element_type=jnp.float32) + b2_ref[...]


def _make_sc_spmm(n_acc, d, chunks, rows_pt):
    epw = chunks * CHUNK
    mesh = plsc.VectorSubcoreMesh(core_axis_name="c", subcore_axis_name="s")

    @functools.partial(
        pl.kernel,
        out_type=[
            jax.ShapeDtypeStruct((NC * n_acc, d), jnp.float32),
            jax.ShapeDtypeStruct((NC * n_acc, d), jnp.float32),
        ],
        mesh=mesh,
        scratch_types=[
            pltpu.VMEM((CHUNK,), jnp.int32),
            pltpu.VMEM((CHUNK,), jnp.int32),
            pltpu.VMEM((CHUNK, d), jnp.float32),
            pltpu.VMEM_SHARED((n_acc, d), jnp.float32),
            pltpu.SemaphoreType.DMA,
        ],
    )
    def spmm(src_hbm, dst_hbm, hspec_hbm, zeros_hbm, ones_hbm, rowidx_hbm,
             hi_out, deg_out, srcv, dstv, rows, acc, sem):
        cid = lax.axis_index("c")
        sid = lax.axis_index("s")
        wid = sid * NC + cid
        row0 = sid * rows_pt
        base = wid * epw
        out0 = cid * n_acc + row0
        nz = rows_pt // CHUNK


        def zero_acc():
            pltpu.sync_copy(zeros_hbm, rows)
            for z in range(nz):
                pltpu.sync_copy(rowidx_hbm.at[pl.ds(row0 + z * CHUNK, CHUNK)],
                                dstv)
                pltpu.sync_copy(rows, acc.at[dstv])
            plsc.subcore_barrier()

        def copy_out(dst_hbm_out):
            for z in range(nz):
                pltpu.sync_copy(rowidx_hbm.at[pl.ds(row0 + z * CHUNK, CHUNK)],
                                dstv)
                pltpu.sync_copy(acc.at[dstv], rows)
                pltpu.sync_copy(rows, dst_hbm_out.at[pl.ds(out0 + z * CHUNK,
                                                           CHUNK)])
            plsc.subcore_barrier()

        zero_acc()

        def step(j, carry):
            off = base + j * CHUNK
            d1 = pltpu.async_copy(src_hbm.at[pl.ds(off, CHUNK)], srcv, sem)
            d2 = pltpu.async_copy(dst_hbm.at[pl.ds(off, CHUNK)], dstv, sem)
            d1.wait()
            d2.wait()
            pltpu.async_copy(hspec_hbm.at[srcv], rows, sem).wait()
            pltpu.sync_copy(rows, acc.at[dstv], add=True)
            return carry

        lax.fori_loop(0, chunks, step, 0)
        plsc.subcore_barrier()
        copy_out(hi_out)

        zero_acc()
        pltpu.sync_copy(ones_hbm, rows)

        def dstep(j, carry):
            off = base + j * CHUNK
            pltpu.sync_copy(dst_hbm.at[pl.ds(off, CHUNK)], dstv)
            pltpu.sync_copy(rows, acc.at[dstv], add=True)
            return carry

        lax.fori_loop(0, chunks, dstep, 0)
        plsc.subcore_barrier()
        copy_out(deg_out)

    return spmm


def kernel(x, e, u, edge_index, W_feat, b_feat, W_eig, b_eig, W_dec, b_dec,
           spec_weight, W_gc, W1, b1, W2, b2):
    n, d = x.shape
    k = e.shape[0]
    hid = W_eig.shape[1]
    m = W_dec.shape[1]
    num_e = edge_index.shape[1]
    nclass = W2.shape[1]
    hdim = W1.shape[1]

    blk = 1000
    nblk = n // blk

    ecol = e.reshape(k, 1)
    div = jnp.exp(jnp.arange(0, hid, 2, dtype=jnp.float32)
                  * (-math.log(10000.0) / hid)).reshape(1, hid // 2)
    We0 = W_eig[0].reshape(1, hid)
    Wsin = W_eig[1:1 + hid // 2]
    Wcos = W_eig[1 + hid // 2:]
    sw0 = spec_weight[0, 0].reshape(1, d)
    swr = spec_weight[0, 1:]

    f32 = jnp.float32
    h, S = pl.pallas_call(
        _pre_kernel,
        grid=(nblk,),
        in_specs=[
            pl.BlockSpec((blk, d), lambda i: (i, 0)),
            pl.BlockSpec((blk, k), lambda i: (i, 0)),
            pl.BlockSpec((d, d), lambda i: (0, 0)),
            pl.BlockSpec((1, d), lambda i: (0, 0)),
            pl.BlockSpec((k, 1), lambda i: (0, 0)),
            pl.BlockSpec((1, hid // 2), lambda i: (0, 0)),
            pl.BlockSpec((1, hid), lambda i: (0, 0)),
            pl.BlockSpec((hid // 2, hid), lambda i: (0, 0)),
            pl.BlockSpec((hid // 2, hid), lambda i: (0, 0)),
            pl.BlockSpec((1, hid), lambda i: (0, 0)),
            pl.BlockSpec((hid, m), lambda i: (0, 0)),
            pl.BlockSpec((1, m), lambda i: (0, 0)),
            pl.BlockSpec((m, d), lambda i: (0, 0)),
        ],
        out_specs=[
            pl.BlockSpec((blk, d), lambda i: (i, 0)),
            pl.BlockSpec((k, d), lambda i: (0, 0)),
        ],
        out_shape=[
            jax.ShapeDtypeStruct((n, d), f32),
            jax.ShapeDtypeStruct((k, d), f32),
        ],
        scratch_shapes=[pltpu.VMEM((k, d), f32)],
    )(x, u, W_feat, b_feat.reshape(1, d), ecol, div, We0, Wsin, Wcos,
      b_eig.reshape(1, hid), W_dec, b_dec.reshape(1, m), swr)

    h_spec = pl.pallas_call(
        _hspec_kernel,
        grid=(nblk,),
        in_specs=[
            pl.BlockSpec((blk, d), lambda i: (i, 0)),
            pl.BlockSpec((blk, k), lambda i: (i, 0)),
            pl.BlockSpec((k, d), lambda i: (0, 0)),
            pl.BlockSpec((1, d), lambda i: (0, 0)),
        ],
        out_specs=pl.BlockSpec((blk, d), lambda i: (i, 0)),
        out_shape=jax.ShapeDtypeStruct((n, d), f32),
    )(h, u, S, sw0)

    chunks = -(-num_e // (NW * CHUNK))
    e_pad = NW * chunks * CHUNK
    rows_pt = ((-(-(n + 1) // NS)) + CHUNK - 1) // CHUNK * CHUNK
    n_acc = rows_pt * NS
    src = jnp.concatenate(
        [edge_index[0], jnp.zeros((e_pad - num_e,), jnp.int32)])
    dst = jnp.concatenate(
        [edge_index[1], jnp.full((e_pad - num_e,), n, jnp.int32)])
    zeros_hbm = jnp.zeros((CHUNK, d), f32)
    ones_hbm = jnp.ones((CHUNK, d), f32)
    rowidx_hbm = jnp.arange(n_acc, dtype=jnp.int32)

    spmm = _make_sc_spmm(n_acc, d, chunks, rows_pt)
    hi_p, deg_p = spmm(src, dst, h_spec, zeros_hbm, ones_hbm, rowidx_hbm)
    hi_p = hi_p.reshape(NC, n_acc, d)
    deg_p = deg_p.reshape(NC, n_acc, d)

    logits = pl.pallas_call(
        _post_kernel,
        grid=(nblk,),
        in_specs=[
            pl.BlockSpec((blk, d), lambda i: (i, 0)),
            pl.BlockSpec((NC, blk, d), lambda i: (0, i, 0)),
            pl.BlockSpec((NC, blk, d), lambda i: (0, i, 0)),
            pl.BlockSpec((d, d), lambda i: (0, 0)),
            pl.BlockSpec((d, hdim), lambda i: (0, 0)),
            pl.BlockSpec((1, hdim), lambda i: (0, 0)),
            pl.BlockSpec((hdim, nclass), lambda i: (0, 0)),
            pl.BlockSpec((1, nclass), lambda i: (0, 0)),
        ],
        out_specs=pl.BlockSpec((blk, nclass), lambda i: (i, 0)),
        out_shape=jax.ShapeDtypeStruct((n, nclass), f32),
    )(h, hi_p, deg_p, W_gc, W1, b1.reshape(1, hdim), W2, b2.reshape(1, nclass))

    return logits


# speedup vs baseline: 1.4581x; 1.0327x over previous
import functools
import math

import jax
import jax.numpy as jnp
from jax import lax
from jax.experimental import pallas as pl
from jax.experimental.pallas import tpu as pltpu
from jax.experimental.pallas import tpu_sc as plsc

ALPHA = 0.1
THETA = math.log(0.5 / 1.0 + 1.0)

NC = 2
NS = 16
NW = NC * NS
CHUNK = 128


def _pre_kernel(x_ref, u_ref, Wf_ref, bf_ref, ecol_ref, div_ref, We0_ref,
                Wsin_ref, Wcos_ref, beig_ref, Wdec_ref, bdec_ref, swr_ref,
                h_ref, S_ref, acc_ref):
    i = pl.program_id(0)
    nsteps = pl.num_programs(0)
    h = jnp.dot(x_ref[...], Wf_ref[...], preferred_element_type=jnp.float32)
    h = h + bf_ref[...]
    h_ref[...] = h

    @pl.when(i == 0)
    def _():
        acc_ref[...] = jnp.zeros_like(acc_ref)

    acc_ref[...] += lax.dot_general(
        u_ref[...], h, (((0,), (0,)), ((), ())),
        preferred_element_type=jnp.float32)

    @pl.when(i == nsteps - 1)
    def _():
        ecol = ecol_ref[...]
        pe = (ecol * 100.0) * div_ref[...]
        eig = (ecol * We0_ref[...]
               + jnp.dot(jnp.sin(pe), Wsin_ref[...],
                         preferred_element_type=jnp.float32)
               + jnp.dot(jnp.cos(pe), Wcos_ref[...],
                         preferred_element_type=jnp.float32)
               + beig_ref[...])
        new_e = jnp.dot(eig, Wdec_ref[...],
                        preferred_element_type=jnp.float32) + bdec_ref[...]
        M = new_e.shape[1]
        combo = new_e[:, 0:1] * swr_ref[0:1, :]
        for m in range(1, M):
            combo = combo + new_e[:, m:m + 1] * swr_ref[m:m + 1, :]
        S_ref[...] = acc_ref[...] * combo


def _hspec_kernel(h_ref, u_ref, S_ref, sw0_ref, out_ref):
    out_ref[...] = (sw0_ref[...] * h_ref[...]
                    + jnp.dot(u_ref[...], S_ref[...],
                              preferred_element_type=jnp.float32))


def _post_kernel(h_ref, hip_ref, degp_ref, Wgc_ref, W1_ref, b1_ref, W2_ref,
                 b2_ref, out_ref):
    hi = hip_ref[0] + hip_ref[1]
    deg = degp_ref[0, :, 0:1] + degp_ref[1, :, 0:1]
    hi = hi / jnp.maximum(deg, 1.0)
    support = (1.0 - ALPHA) * hi + ALPHA * h_ref[...]
    out = THETA * jnp.dot(support, Wgc_ref[...],
                          preferred_element_type=jnp.float32) \
        + (1.0 - THETA) * support
    out = jnp.maximum(
        jnp.dot(out, W1_ref[...], preferred_element_type=jnp.float32)
        + b1_ref[...], 0.0)
    out_ref[...] = jnp.dot(out, W2_ref[...],
                           preferred_element_type=jnp.float32) + b2_ref[...]


def _make_sc_spmm(n_acc, d, chunks, rows_pt):
    epw = chunks * CHUNK
    mesh = plsc.VectorSubcoreMesh(core_axis_name="c", subcore_axis_name="s")

    @functools.partial(
        pl.kernel,
        out_type=[
            jax.ShapeDtypeStruct((NC * n_acc, d), jnp.float32),
            jax.ShapeDtypeStruct((NC * n_acc, d), jnp.float32),
        ],
        mesh=mesh,
        scratch_types=[
            pltpu.VMEM((CHUNK,), jnp.int32),
            pltpu.VMEM((CHUNK,), jnp.int32),
            pltpu.VMEM((CHUNK,), jnp.int32),
            pltpu.VMEM((CHUNK, d), jnp.float32),
            pltpu.VMEM_SHARED((n_acc, d), jnp.float32),
            pltpu.SemaphoreType.DMA,
            pltpu.SemaphoreType.DMA,
        ],
    )
    def spmm(src_hbm, dst_hbm, hspec_hbm, zeros_hbm, ones_hbm, rowidx_hbm,
             hi_out, deg_out, srcv, dstv, dstvB, rows, acc, sem, semB):
        cid = lax.axis_index("c")
        sid = lax.axis_index("s")
        wid = sid * NC + cid
        row0 = sid * rows_pt
        base = wid * epw
        out0 = cid * n_acc + row0
        nz = rows_pt // CHUNK


        def zero_acc():
            pltpu.sync_copy(zeros_hbm, rows)
            for z in range(nz):
                pltpu.sync_copy(rowidx_hbm.at[pl.ds(row0 + z * CHUNK, CHUNK)],
                                dstv)
                pltpu.sync_copy(rows, acc.at[dstv])
            plsc.subcore_barrier()

        def copy_out(dst_hbm_out):
            for z in range(nz):
                pltpu.sync_copy(rowidx_hbm.at[pl.ds(row0 + z * CHUNK, CHUNK)],
                                dstv)
                pltpu.sync_copy(acc.at[dstv], rows)
                pltpu.sync_copy(rows, dst_hbm_out.at[pl.ds(out0 + z * CHUNK,
                                                           CHUNK)])
            plsc.subcore_barrier()

        zero_acc()

        def step(j, carry):
            off = base + j * CHUNK
            d1 = pltpu.async_copy(src_hbm.at[pl.ds(off, CHUNK)], srcv, sem)
            d2 = pltpu.async_copy(dst_hbm.at[pl.ds(off, CHUNK)], dstv, sem)
            d1.wait()
            d2.wait()
            pltpu.async_copy(hspec_hbm.at[srcv], rows, sem).wait()
            pltpu.sync_copy(rows, acc.at[dstv], add=True)
            return carry

        lax.fori_loop(0, chunks, step, 0)
        plsc.subcore_barrier()
        copy_out(hi_out)

        zero_acc()
        pltpu.sync_copy(ones_hbm, rows)

        def dstep(i, carry):
            j = 2 * i
            off = base + j * CHUNK
            d1 = pltpu.async_copy(dst_hbm.at[pl.ds(off, CHUNK)], dstv, sem)
            d2 = pltpu.async_copy(dst_hbm.at[pl.ds(off + CHUNK, CHUNK)],
                                  dstvB, semB)
            d1.wait()
            pltpu.sync_copy(rows, acc.at[dstv], add=True)
            d2.wait()
            pltpu.sync_copy(rows, acc.at[dstvB], add=True)
            return carry

        lax.fori_loop(0, chunks // 2, dstep, 0)
        if chunks % 2:
            off = base + (chunks - 1) * CHUNK
            pltpu.sync_copy(dst_hbm.at[pl.ds(off, CHUNK)], dstv)
            pltpu.sync_copy(rows, acc.at[dstv], add=True)
        plsc.subcore_barrier()
        copy_out(deg_out)

    return spmm


def kernel(x, e, u, edge_index, W_feat, b_feat, W_eig, b_eig, W_dec, b_dec,
           spec_weight, W_gc, W1, b1, W2, b2):
    n, d = x.shape
    k = e.shape[0]
    hid = W_eig.shape[1]
    m = W_dec.shape[1]
    num_e = edge_index.shape[1]
    nclass = W2.shape[1]
    hdim = W1.shape[1]

    blk = 1000
    nblk = n // blk

    ecol = e.reshape(k, 1)
    div = jnp.exp(jnp.arange(0, hid, 2, dtype=jnp.float32)
                  * (-math.log(10000.0) / hid)).reshape(1, hid // 2)
    We0 = W_eig[0].reshape(1, hid)
    Wsin = W_eig[1:1 + hid // 2]
    Wcos = W_eig[1 + hid // 2:]
    sw0 = spec_weight[0, 0].reshape(1, d)
    swr = spec_weight[0, 1:]

    f32 = jnp.float32
    h, S = pl.pallas_call(
        _pre_kernel,
        grid=(nblk,),
        in_specs=[
            pl.BlockSpec((blk, d), lambda i: (i, 0)),
            pl.BlockSpec((blk, k), lambda i: (i, 0)),
            pl.BlockSpec((d, d), lambda i: (0, 0)),
            pl.BlockSpec((1, d), lambda i: (0, 0)),
            pl.BlockSpec((k, 1), lambda i: (0, 0)),
            pl.BlockSpec((1, hid // 2), lambda i: (0, 0)),
            pl.BlockSpec((1, hid), lambda i: (0, 0)),
            pl.BlockSpec((hid // 2, hid), lambda i: (0, 0)),
            pl.BlockSpec((hid // 2, hid), lambda i: (0, 0)),
            pl.BlockSpec((1, hid), lambda i: (0, 0)),
            pl.BlockSpec((hid, m), lambda i: (0, 0)),
            pl.BlockSpec((1, m), lambda i: (0, 0)),
            pl.BlockSpec((m, d), lambda i: (0, 0)),
        ],
        out_specs=[
            pl.BlockSpec((blk, d), lambda i: (i, 0)),
            pl.BlockSpec((k, d), lambda i: (0, 0)),
        ],
        out_shape=[
            jax.ShapeDtypeStruct((n, d), f32),
            jax.ShapeDtypeStruct((k, d), f32),
        ],
        scratch_shapes=[pltpu.VMEM((k, d), f32)],
    )(x, u, W_feat, b_feat.reshape(1, d), ecol, div, We0, Wsin, Wcos,
      b_eig.reshape(1, hid), W_dec, b_dec.reshape(1, m), swr)

    h_spec = pl.pallas_call(
        _hspec_kernel,
        grid=(nblk,),
        in_specs=[
            pl.BlockSpec((blk, d), lambda i: (i, 0)),
            pl.BlockSpec((blk, k), lambda i: (i, 0)),
            pl.BlockSpec((k, d), lambda i: (0, 0)),
            pl.BlockSpec((1, d), lambda i: (0, 0)),
        ],
        out_specs=pl.BlockSpec((blk, d), lambda i: (i, 0)),
        out_shape=jax.ShapeDtypeStruct((n, d), f32),
    )(h, u, S, sw0)

    chunks = -(-num_e // (NW * CHUNK))
    e_pad = NW * chunks * CHUNK
    rows_pt = ((-(-(n + 1) // NS)) + CHUNK - 1) // CHUNK * CHUNK
    n_acc = rows_pt * NS
    src = jnp.concatenate(
        [edge_index[0], jnp.zeros((e_pad - num_e,), jnp.int32)])
    dst = jnp.concatenate(
        [edge_index[1], jnp.full((e_pad - num_e,), n, jnp.int32)])
    zeros_hbm = jnp.zeros((CHUNK, d), f32)
    ones_hbm = jnp.ones((CHUNK, d), f32)
    rowidx_hbm = jnp.arange(n_acc, dtype=jnp.int32)

    spmm = _make_sc_spmm(n_acc, d, chunks, rows_pt)
    hi_p, deg_p = spmm(src, dst, h_spec, zeros_hbm, ones_hbm, rowidx_hbm)
    hi_p = hi_p.reshape(NC, n_acc, d)
    deg_p = deg_p.reshape(NC, n_acc, d)

    logits = pl.pallas_call(
        _post_kernel,
        grid=(nblk,),
        in_specs=[
            pl.BlockSpec((blk, d), lambda i: (i, 0)),
            pl.BlockSpec((NC, blk, d), lambda i: (0, i, 0)),
            pl.BlockSpec((NC, blk, d), lambda i: (0, i, 0)),
            pl.BlockSpec((d, d), lambda i: (0, 0)),
            pl.BlockSpec((d, hdim), lambda i: (0, 0)),
            pl.BlockSpec((1, hdim), lambda i: (0, 0)),
            pl.BlockSpec((hdim, nclass), lambda i: (0, 0)),
            pl.BlockSpec((1, nclass), lambda i: (0, 0)),
        ],
        out_specs=pl.BlockSpec((blk, nclass), lambda i: (i, 0)),
        out_shape=jax.ShapeDtypeStruct((n, nclass), f32),
    )(h, hi_p, deg_p, W_gc, W1, b1.reshape(1, hdim), W2, b2.reshape(1, nclass))

    return logits
